# baseline (speedup 1.0000x reference)
import functools

import jax
import jax.numpy as jnp
from jax import lax
from jax.experimental import pallas as pl
from jax.experimental.pallas import tpu as pltpu
from jax.experimental.pallas import tpu_sc as plsc

_B, _N, _D, _K = 8, 576, 64, 1024
_ROWS = _B * _N
_RB = 576
_G = _ROWS // _RB


def _tc_body(z_ref, zsq_ref, cbt_ref, idx_ref, loss_ref):
    i = pl.program_id(0)
    zb = z_ref[...]
    cbt = cbt_ref[...]
    dot = lax.dot_general(zb, cbt, (((1,), (0,)), ((), ())),
                          preferred_element_type=jnp.float32)
    z_sq = zsq_ref[...]
    c_sq = jnp.sum(cbt * cbt, axis=0, keepdims=True)
    d2 = (z_sq - 2.0 * dot) + c_sq
    dist = jnp.sqrt(jnp.maximum(d2, 0.0))
    mval = jnp.min(dist, axis=1, keepdims=True)
    iota = lax.broadcasted_iota(jnp.int32, (_RB, _K), 1)
    idxc = jnp.min(jnp.where(dist == mval, iota, _K), axis=1,
                   keepdims=True)
    idx_ref[...] = idxc
    part = jnp.sum(mval * mval)

    @pl.when(i == 0)
    def _init():
        loss_ref[0, 0] = 0.0

    loss_ref[0, 0] = loss_ref[0, 0] + part

    @pl.when(i == _G - 1)
    def _fin():
        loss_ref[0, 0] = loss_ref[0, 0] * (1.0 / (_ROWS * _D))


def _tc_call(zf, zsq, cbt):
    return pl.pallas_call(
        _tc_body,
        grid=(_G,),
        in_specs=[
            pl.BlockSpec((_RB, _D), lambda i: (i, 0)),
            pl.BlockSpec((_RB, 1), lambda i: (i, 0)),
            pl.BlockSpec((_D, _K), lambda i: (0, 0)),
        ],
        out_specs=[
            pl.BlockSpec((_RB, 1), lambda i: (i, 0)),
            pl.BlockSpec(memory_space=pltpu.SMEM),
        ],
        out_shape=[
            jax.ShapeDtypeStruct((_ROWS, 1), jnp.int32),
            jax.ShapeDtypeStruct((1, 1), jnp.float32),
        ],
    )(zf, zsq, cbt)


@functools.cache
def _sc_gather_kernel():
    info = plsc.get_sparse_core_info()
    nc, ns = info.num_cores, info.num_subcores
    nw = nc * ns
    bpw = _ROWS // nw
    ch = 72
    nch = bpw // ch

    @functools.partial(
        pl.kernel,
        out_type=jax.ShapeDtypeStruct((_ROWS, 128), jnp.float32),
        mesh=plsc.VectorSubcoreMesh(core_axis_name="c", subcore_axis_name="s"),
        scratch_types=[
            pltpu.VMEM((nch, ch), jnp.int32),
            pltpu.VMEM((nch, ch, 128), jnp.float32),
            pltpu.SemaphoreType.DMA,
        ],
    )
    def _sc_gather(cb_hbm, idx_hbm, out_hbm, idx_v, rows_v, sem):
        wid = lax.axis_index("s") * nc + lax.axis_index("c")
        base = wid * bpw
        for j in range(nch):
            pltpu.sync_copy(idx_hbm.at[pl.ds(base + j * ch, ch)], idx_v.at[j])
        cps = [pltpu.async_copy(cb_hbm.at[idx_v.at[j]], rows_v.at[j], sem)
               for j in range(nch)]
        for cp in cps:
            cp.wait()
        for j in range(nch):
            pltpu.sync_copy(rows_v.at[j], out_hbm.at[pl.ds(base + j * ch, ch)])

    return _sc_gather


def kernel(z, codebook):
    zf = z.reshape(_ROWS, _D)
    zsq = jnp.sum(z ** 2, axis=-1, keepdims=True).reshape(_ROWS, 1)
    idx2, loss2 = _tc_call(zf, zsq, codebook.T)
    idxf = idx2.reshape(_ROWS)
    cb_pad = jnp.pad(codebook, ((0, 0), (0, 128 - _D)))
    zq = _sc_gather_kernel()(cb_pad, idxf)
    z_q = zq[:, :_D].reshape(_B, _N, _D)
    out = z + (z_q - z)
    loss = loss2.reshape(())
    return (out, loss)
